# Initial kernel scaffold; baseline (speedup 1.0000x reference)
import jax
import jax.numpy as jnp
from jax.experimental import pallas as pl


def kernel(x, wg, W, b):
    raise NotImplementedError("write your pallas kernel here")


# speedup vs baseline: 1.7054x; 1.7054x over previous
import functools

import jax
import jax.numpy as jnp
from jax.experimental import pallas as pl
from jax.experimental.pallas import tpu as pltpu

_CS_BLK = 512
_TOK_BLK = 512


def _gating_kernel(x_ref, wg_ref, u_ref, coef_ref, laux_ref, *, S, E, CAP):
    logits = jax.lax.dot_general(
        wg_ref[...], x_ref[...],
        dimension_numbers=(((0,), (1,)), ((), ())),
        preferred_element_type=jnp.float32,
    )

    m = jnp.max(logits, axis=0, keepdims=True)
    ex = jnp.exp(logits - m)
    gates = ex / jnp.sum(ex, axis=0, keepdims=True)

    iota = jax.lax.broadcasted_iota(jnp.int32, (E, S), 0)
    big = jnp.int32(E)

    gmax = jnp.max(gates, axis=0, keepdims=True)
    idx1 = jnp.min(jnp.where(gates == gmax, iota, big), axis=0, keepdims=True)
    mask1 = (iota == idx1).astype(jnp.float32)

    neg_inf = jnp.float32(-jnp.inf)
    logits2 = jnp.where(mask1 > 0, neg_inf, logits)
    lmax = jnp.max(logits2, axis=0, keepdims=True)
    idx2 = jnp.min(jnp.where(logits2 == lmax, iota, big), axis=0, keepdims=True)
    mask2 = (iota == idx2).astype(jnp.float32)

    mm = jnp.concatenate([mask1, mask2], axis=0)
    U = u_ref[...]
    R = U.shape[0]
    carry = jnp.zeros((2 * E, 1), jnp.float32)
    blocks = []
    for i in range(S // R):
        blk = mm[:, i * R:(i + 1) * R]
        csb = jax.lax.dot_general(
            blk, U, dimension_numbers=(((1,), (0,)), ((), ())),
            preferred_element_type=jnp.float32,
        ) + carry
        blocks.append(csb)
        carry = csb[:, R - 1:R]
    cs = jnp.concatenate(blocks, axis=1)

    count1 = carry[:E]
    loc1 = cs[:E] - 1.0
    loc2 = cs[E:] - 1.0 + count1

    me = jnp.mean(gates, axis=1, keepdims=True)
    ce = jnp.mean(mask1, axis=1, keepdims=True)
    laux_ref[...] = jnp.sum(me * ce, axis=0, keepdims=True) * jnp.float32(E)

    cap = jnp.float32(CAP)
    mask1k = mask1 * (loc1 < cap).astype(jnp.float32)
    mask2k = mask2 * (loc2 < cap).astype(jnp.float32)

    g1 = jnp.sum(gates * mask1k, axis=0, keepdims=True)
    g2 = jnp.sum(gates * mask2k, axis=0, keepdims=True)
    denom = jnp.maximum(g1 + g2, jnp.float32(1e-9))
    g1 = g1 / denom
    g2 = g2 / denom

    coef_t = g1 * mask1k + g2 * mask2k
    coef_ref[...] = coef_t.T


def _expert_kernel(x_ref, w_ref, coef_ref, b_ref, o_ref, *, E):
    xb = x_ref[...]
    coef = coef_ref[...]
    acc = jax.lax.dot_general(
        coef, b_ref[...], dimension_numbers=(((1,), (0,)), ((), ())),
        preferred_element_type=jnp.float32,
        precision=jax.lax.Precision.HIGHEST,
    )
    for e in range(E):
        y = jax.lax.dot_general(
            xb, w_ref[e], dimension_numbers=(((1,), (0,)), ((), ())),
            preferred_element_type=jnp.float32,
        )
        acc = acc + coef[:, e:e + 1] * y
    o_ref[...] = acc


def kernel(x, wg, W, b):
    B, T, D = x.shape
    E = wg.shape[1]
    S = B * T
    CAP = 2 * S // E
    R = _CS_BLK
    M = _TOK_BLK

    xr = x.reshape(S, D)
    U = jnp.triu(jnp.ones((R, R), jnp.float32))

    coef, laux = pl.pallas_call(
        functools.partial(_gating_kernel, S=S, E=E, CAP=CAP),
        out_shape=[
            jax.ShapeDtypeStruct((S, E), jnp.float32),
            jax.ShapeDtypeStruct((1, 1), jnp.float32),
        ],
    )(xr, wg, U)

    xb16 = xr.astype(jnp.bfloat16)
    Wb16 = W.astype(jnp.bfloat16)

    out = pl.pallas_call(
        functools.partial(_expert_kernel, E=E),
        grid=(S // M,),
        in_specs=[
            pl.BlockSpec((M, D), lambda i: (i, 0)),
            pl.BlockSpec((E, D, D), lambda i: (0, 0, 0)),
            pl.BlockSpec((M, E), lambda i: (i, 0)),
            pl.BlockSpec((E, D), lambda i: (0, 0)),
        ],
        out_specs=pl.BlockSpec((M, D), lambda i: (i, 0)),
        out_shape=jax.ShapeDtypeStruct((S, D), jnp.float32),
    )(xb16, Wb16, coef, b)

    return out.reshape(B, T, D), laux[0, 0]
